# baseline (speedup 1.0000x reference)
import functools

import jax
import jax.numpy as jnp
from jax import lax
from jax.experimental import pallas as pl
from jax.experimental.pallas import tpu as pltpu
from jax.experimental.pallas import tpu_sc as plsc

B = 4096
F = 26
V = 100000
D = 64

NC = 2
NS = 16
NW = NC * NS
CPW = D // NW
W = 100096

_mesh = plsc.VectorSubcoreMesh(core_axis_name="c", subcore_axis_name="s")


@functools.partial(
    pl.kernel,
    mesh=_mesh,
    compiler_params=pltpu.CompilerParams(
        needs_layout_passes=False, use_tc_tiling_on_sc=True
    ),
    out_type=jax.ShapeDtypeStruct((F * D, B), jnp.float32),
    scratch_types=[
        pltpu.VMEM((W,), jnp.float32),
        pltpu.VMEM((B,), jnp.int32),
        pltpu.VMEM((2 * B,), jnp.float32),
        pltpu.SemaphoreType.DMA,
        pltpu.SemaphoreType.DMA,
        pltpu.SemaphoreType.DMA,
        pltpu.SemaphoreType.DMA,
    ],
)
def _sc_gather(
    tableT_hbm, values_hbm, out_hbm, stage_v, vals_v, col_v,
    hsem0, hsem1, vsem, wsem,
):
    hsems = (hsem0, hsem0, hsem1, hsem1)
    wid = lax.axis_index("s") * NC + lax.axis_index("c")
    iota = lax.broadcasted_iota(jnp.int32, (16,), 0)
    H = W // 2
    NU = F * CPW

    QS = (25088, 25088, 24960, 24960)
    QO = (0, 25088, 50176, 75136)

    def stage(u, q):
        f, j = divmod(u, CPW)
        c = wid * CPW + j
        lo = f * V - (f * V) % 128
        lo_t = pl.multiple_of(wid * 0 + lo + QO[q], 128)
        pltpu.async_copy(
            tableT_hbm.at[c, pl.ds(lo_t, QS[q])],
            stage_v.at[pl.ds(QO[q], QS[q])],
            hsems[q],
        )

    def wait_stage(q):
        pltpu.make_async_copy(
            tableT_hbm.at[0, pl.ds(0, QS[q])],
            stage_v.at[pl.ds(QO[q], QS[q])],
            hsems[q],
        ).wait()

    for q in range(4):
        stage(0, q)
    for u in range(NU):
        f, j = divmod(u, CPW)
        c = wid * CPW + j
        fv = f * V
        off = fv % 128
        ub = u % 2

        if j == 0:
            pltpu.async_copy(
                values_hbm.at[pl.ds(f * B, B)], vals_v, vsem
            ).wait()

        if u >= 2:
            pltpu.make_async_copy(
                col_v.at[pl.ds(ub * B, B)], out_hbm.at[0, :], wsem
            ).wait()

        for p in range(2):
            wait_stage(2 * p)
            wait_stage(2 * p + 1)

            def gather(g, carry, ub=ub, off=off, p=p):
                idx = vals_v[pl.ds(g * 16, 16)] + off
                m = (idx < QO[2]) if p == 0 else (idx >= QO[2])
                val = plsc.load_gather(stage_v, [idx], mask=m)
                plsc.store_scatter(
                    col_v, [ub * B + g * 16 + iota], val, mask=m
                )
                return carry

            lax.fori_loop(0, B // 16, gather, 0)

            if u + 1 < NU:
                stage(u + 1, 2 * p)
                stage(u + 1, 2 * p + 1)

        pltpu.async_copy(col_v.at[pl.ds(ub * B, B)], out_hbm.at[f * D + c, :], wsem)

    pltpu.make_async_copy(col_v.at[pl.ds(0, B)], out_hbm.at[0, :], wsem).wait()
    pltpu.make_async_copy(col_v.at[pl.ds(B, B)], out_hbm.at[0, :], wsem).wait()


def kernel(values, batch_offsets, table):
    del batch_offsets
    out2d = _sc_gather(table.T, values.reshape(-1))
    return out2d.reshape(F, D, B).transpose(2, 0, 1)


# speedup vs baseline: 1.0401x; 1.0401x over previous
import functools

import jax
import jax.numpy as jnp
from jax import lax
from jax.experimental import pallas as pl
from jax.experimental.pallas import tpu as pltpu
from jax.experimental.pallas import tpu_sc as plsc

B = 4096
F = 26
V = 100000
D = 64

NC = 2
NS = 16
NW = NC * NS
CPW = D // NW
W = 100096

_mesh = plsc.VectorSubcoreMesh(core_axis_name="c", subcore_axis_name="s")


@functools.partial(
    pl.kernel,
    mesh=_mesh,
    compiler_params=pltpu.CompilerParams(
        needs_layout_passes=False, use_tc_tiling_on_sc=True
    ),
    out_type=jax.ShapeDtypeStruct((F * D, B), jnp.float32),
    scratch_types=[
        pltpu.VMEM((W,), jnp.float32),
        pltpu.VMEM((B,), jnp.int32),
        pltpu.VMEM((2 * B,), jnp.float32),
        pltpu.SemaphoreType.DMA,
        pltpu.SemaphoreType.DMA,
    ],
)
def _sc_gather(tableT_hbm, values_hbm, out_hbm, stage_v, vals_v, col_v, gsem, wsem):
    wid = lax.axis_index("s") * NC + lax.axis_index("c")
    iota = lax.broadcasted_iota(jnp.int32, (16,), 0)
    H = W // 2
    NU = F * CPW

    def piece_of(k):
        u, p = divmod(k, 2)
        f, j = divmod(u, CPW)
        c = wid * CPW + j
        fv = f * V
        off = fv % 128
        lo = fv - off
        return u, p, f, j, c, off, lo

    def stage(k):
        u, p, f, j, c, off, lo = piece_of(k)
        lo_t = pl.multiple_of(wid * 0 + lo + p * H, 128)
        pltpu.async_copy(
            tableT_hbm.at[c, pl.ds(lo_t, H)], stage_v.at[pl.ds((k % 2) * H, H)], gsem
        )

    def wait_stage(k):
        pltpu.make_async_copy(
            tableT_hbm.at[0, pl.ds(0, H)], stage_v.at[pl.ds((k % 2) * H, H)], gsem
        ).wait()

    stage(0)
    for u in range(NU):
        f, j = divmod(u, CPW)
        c = wid * CPW + j
        fv = f * V
        off = fv % 128
        ub = u % 2

        if j == 0:
            pltpu.async_copy(
                values_hbm.at[pl.ds(f * B, B)], vals_v, gsem
            ).wait()

        if u >= 2:
            pltpu.make_async_copy(
                col_v.at[pl.ds(ub * B, B)], out_hbm.at[0, :], wsem
            ).wait()

        for p in range(2):
            k = u * 2 + p
            wait_stage(k)
            if k + 1 < NU * 2:
                stage(k + 1)

            def gather(g, carry, ub=ub, off=off, p=p, kb=k % 2):
                idx = vals_v[pl.ds(g * 16, 16)] + off
                loc = idx - p * H
                m = (loc < H) if p == 0 else (loc >= 0)
                val = plsc.load_gather(stage_v, [loc + kb * H], mask=m)
                plsc.store_scatter(
                    col_v, [ub * B + g * 16 + iota], val, mask=m
                )
                return carry

            lax.fori_loop(0, B // 16, gather, 0, unroll=8)

        pltpu.async_copy(col_v.at[pl.ds(ub * B, B)], out_hbm.at[f * D + c, :], wsem)

    pltpu.make_async_copy(col_v.at[pl.ds(0, B)], out_hbm.at[0, :], wsem).wait()
    pltpu.make_async_copy(col_v.at[pl.ds(B, B)], out_hbm.at[0, :], wsem).wait()


def kernel(values, batch_offsets, table):
    del batch_offsets
    out2d = _sc_gather(table.T, values.reshape(-1))
    return out2d.reshape(F, D, B).transpose(2, 0, 1)
